# baseline (speedup 1.0000x reference)
import jax
import jax.numpy as jnp
from jax import lax
from jax.experimental import pallas as pl
from jax.experimental.pallas import tpu as pltpu, tpu_sc as plsc


def kernel(ids, table):
    batch = ids.shape[0]
    dim = table.shape[1]
    info = plsc.get_sparse_core_info()
    num_cores = info.num_cores
    nw = num_cores * info.num_subcores
    bpw = batch // nw

    nb = 8
    chunk = bpw // nb

    mesh = plsc.VectorSubcoreMesh(core_axis_name="c", subcore_axis_name="s")
    ids32 = ids.astype(jnp.int32)

    def body(ids_hbm, table_hbm, out_hbm, idx_v, rows_v, gsem, wsem):
        wid = lax.axis_index("s") * num_cores + lax.axis_index("c")
        base = wid * bpw
        pltpu.sync_copy(ids_hbm.at[pl.ds(base, bpw)], idx_v)

        def gather(b):
            return pltpu.async_copy(
                table_hbm.at[idx_v.at[pl.ds(b * chunk, chunk)]],
                rows_v.at[pl.ds(b * chunk, chunk)], gsem.at[b])

        def write(b):
            return pltpu.async_copy(
                rows_v.at[pl.ds(b * chunk, chunk)],
                out_hbm.at[pl.ds(base + b * chunk, chunk)], wsem.at[b])

        gathers = [gather(0), gather(1)]
        writes = []
        for b in range(nb):
            gathers[b].wait()
            writes.append(write(b))
            if b + 2 < nb:
                gathers.append(gather(b + 2))
        for w in writes:
            w.wait()

    f = pl.kernel(
        body,
        out_type=jax.ShapeDtypeStruct((batch, dim), jnp.float32),
        mesh=mesh,
        scratch_types=[
            pltpu.VMEM((bpw,), jnp.int32),
            pltpu.VMEM((bpw, dim), jnp.float32),
            pltpu.SemaphoreType.DMA((nb,)),
            pltpu.SemaphoreType.DMA((nb,)),
        ],
    )
    return f(ids32, table)


# speedup vs baseline: 1.0707x; 1.0707x over previous
import jax
import jax.numpy as jnp
from jax import lax
from jax.experimental import pallas as pl
from jax.experimental.pallas import tpu as pltpu, tpu_sc as plsc


def kernel(ids, table):
    batch = ids.shape[0]
    dim = table.shape[1]
    info = plsc.get_sparse_core_info()
    num_cores = info.num_cores
    nw = num_cores * info.num_subcores
    bpw = batch // nw

    mesh = plsc.VectorSubcoreMesh(core_axis_name="c", subcore_axis_name="s")
    ids32 = ids.astype(jnp.int32)

    def body(ids_hbm, table_hbm, out_hbm, idx_v, rows_v, sem):
        wid = lax.axis_index("s") * num_cores + lax.axis_index("c")
        base = wid * bpw
        pltpu.sync_copy(ids_hbm.at[pl.ds(base, bpw)], idx_v)
        pltpu.async_copy(table_hbm.at[idx_v], rows_v, sem).wait()
        pltpu.sync_copy(rows_v, out_hbm.at[pl.ds(base, bpw)])

    f = pl.kernel(
        body,
        out_type=jax.ShapeDtypeStruct((batch, dim), jnp.float32),
        mesh=mesh,
        scratch_types=[
            pltpu.VMEM((bpw,), jnp.int32),
            pltpu.VMEM((bpw, dim), jnp.float32),
            pltpu.SemaphoreType.DMA,
        ],
    )
    return f(ids32, table)


# speedup vs baseline: 1.0728x; 1.0019x over previous
import jax
import jax.numpy as jnp
from jax import lax
from jax.experimental import pallas as pl
from jax.experimental.pallas import tpu as pltpu, tpu_sc as plsc


def kernel(ids, table):
    batch = ids.shape[0]
    dim = table.shape[1]
    info = plsc.get_sparse_core_info()
    num_cores = info.num_cores
    nw = num_cores * info.num_subcores
    bpw = batch // nw

    mesh = plsc.VectorSubcoreMesh(core_axis_name="c", subcore_axis_name="s")
    ids32 = ids.astype(jnp.int32)

    half = bpw // 2

    def body(ids_hbm, table_hbm, out_hbm, idx_v, rows_v, isem, gsem, wsem):
        wid = lax.axis_index("s") * num_cores + lax.axis_index("c")
        base = wid * bpw
        ic = [pltpu.async_copy(ids_hbm.at[pl.ds(base + h * half, half)],
                               idx_v.at[pl.ds(h * half, half)], isem.at[h])
              for h in range(2)]
        gathers = []
        for h in range(2):
            ic[h].wait()
            gathers.append(pltpu.async_copy(
                table_hbm.at[idx_v.at[pl.ds(h * half, half)]],
                rows_v.at[pl.ds(h * half, half)], gsem.at[h]))
        writes = []
        for h in range(2):
            gathers[h].wait()
            writes.append(pltpu.async_copy(
                rows_v.at[pl.ds(h * half, half)],
                out_hbm.at[pl.ds(base + h * half, half)], wsem.at[h]))
        for w in writes:
            w.wait()

    f = pl.kernel(
        body,
        out_type=jax.ShapeDtypeStruct((batch, dim), jnp.float32),
        mesh=mesh,
        scratch_types=[
            pltpu.VMEM((bpw,), jnp.int32),
            pltpu.VMEM((bpw, dim), jnp.float32),
            pltpu.SemaphoreType.DMA((2,)),
            pltpu.SemaphoreType.DMA((2,)),
            pltpu.SemaphoreType.DMA((2,)),
        ],
    )
    return f(ids32, table)
